# Initial kernel scaffold; baseline (speedup 1.0000x reference)
import jax
import jax.numpy as jnp
from jax.experimental import pallas as pl


def kernel(item_ids, pos_ids, action_ids, user_ids, item_table, user_table, pos_table, action_table):
    raise NotImplementedError("write your pallas kernel here")


# speedup vs baseline: 5.3703x; 5.3703x over previous
import functools

import jax
import jax.numpy as jnp
from jax import lax
from jax.experimental import pallas as pl
from jax.experimental.pallas import tpu as pltpu
from jax.experimental.pallas import tpu_sc as plsc

B, L, D = 4096, 200, 64
POS_V = 2 * 200 + 1
ACT_V = 8 + 1
NC, NS = 2, 16
NW = NC * NS
BPW = B // NW
CB = 32
NCHUNK = BPW // CB
G0, G1 = 104, 96
NG = D // 16


def _sc_body(item_ids, pos_ids, act_ids, user_ids,
             item_tab, user_tab, pos_tab, act_tab,
             out_hbm,
             pos_loc, act_loc, uid_loc, user_loc,
             iid_loc, pid_loc, aid_loc, rows, gsem):
    cid = lax.axis_index("c")
    sid = lax.axis_index("s")
    wid = sid * NC + cid
    base_b = wid * BPW

    pltpu.sync_copy(pos_tab, pos_loc)
    pltpu.sync_copy(act_tab, act_loc)
    pltpu.sync_copy(user_ids.at[pl.ds(base_b, BPW)], uid_loc)
    pltpu.async_copy(user_tab.at[uid_loc], user_loc, gsem).wait()

    def chunk_body(c, carry):
        cb0 = base_b + c * CB
        pltpu.sync_copy(item_ids.at[pl.ds(cb0, CB)], iid_loc)
        pltpu.sync_copy(pos_ids.at[pl.ds(cb0, CB)], pid_loc)
        pltpu.sync_copy(act_ids.at[pl.ds(cb0, CB)], aid_loc)

        def block_body(bb, carry2):
            gb = cb0 + bb
            lb = c * CB + bb
            h0 = pltpu.async_copy(item_tab.at[iid_loc.at[bb, pl.ds(0, G0)]],
                                  rows.at[pl.ds(0, G0)], gsem)
            h1 = pltpu.async_copy(item_tab.at[iid_loc.at[bb, pl.ds(G0, G1)]],
                                  rows.at[pl.ds(G0, G1)], gsem)
            h0.wait()
            h1.wait()

            uvs = [user_loc[lb, pl.ds(g * 16, 16)] for g in range(NG)]

            def do_rows(i0, pvec, avec, j0, nrows):
                for j in range(nrows):
                    i = i0 + j
                    pid = pvec[j0 + j]
                    aid = avec[j0 + j]
                    for g in range(NG):
                        sl = pl.ds(g * 16, 16)
                        rows[i, sl] = (rows[i, sl] + pos_loc[pid, sl]
                                       + act_loc[aid, sl] + uvs[g])

            def grp_body(g16, carry3):
                i0 = g16 * 16
                pvec = pid_loc[bb, pl.ds(i0, 16)]
                avec = aid_loc[bb, pl.ds(i0, 16)]
                do_rows(i0, pvec, avec, 0, 16)
                return carry3

            lax.fori_loop(0, L // 16, grp_body, 0)
            ntail = L % 16
            if ntail:
                pvec = pid_loc[bb, pl.ds(L - 16, 16)]
                avec = aid_loc[bb, pl.ds(L - 16, 16)]
                do_rows(L - ntail, pvec, avec, 16 - ntail, ntail)
            pltpu.sync_copy(rows, out_hbm.at[gb])
            return carry2

        lax.fori_loop(0, CB, block_body, 0)
        return carry

    lax.fori_loop(0, NCHUNK, chunk_body, 0)


@functools.partial(jax.jit, static_argnums=())
def _run(item_ids, pos_ids, act_ids, user_ids,
         item_tab, user_tab, pos_tab, act_tab):
    mesh = plsc.VectorSubcoreMesh(core_axis_name="c", subcore_axis_name="s")
    f = functools.partial(
        pl.kernel,
        out_type=jax.ShapeDtypeStruct((B, L, D), jnp.float32),
        mesh=mesh,
        compiler_params=pltpu.CompilerParams(use_tc_tiling_on_sc=False),
        scratch_types=[
            pltpu.VMEM((POS_V, D), jnp.float32),
            pltpu.VMEM((ACT_V, D), jnp.float32),
            pltpu.VMEM((BPW,), jnp.int32),
            pltpu.VMEM((BPW, D), jnp.float32),
            pltpu.VMEM((CB, L), jnp.int32),
            pltpu.VMEM((CB, L), jnp.int32),
            pltpu.VMEM((CB, L), jnp.int32),
            pltpu.VMEM((L, D), jnp.float32),
            pltpu.SemaphoreType.DMA,
        ],
    )(_sc_body)
    return f(item_ids, pos_ids, act_ids, user_ids,
             item_tab, user_tab, pos_tab, act_tab)


def kernel(item_ids, pos_ids, action_ids, user_ids,
           item_table, user_table, pos_table, action_table):
    return _run(item_ids.astype(jnp.int32), pos_ids.astype(jnp.int32),
                action_ids.astype(jnp.int32), user_ids.astype(jnp.int32),
                item_table, user_table, pos_table, action_table)


# speedup vs baseline: 10.1200x; 1.8844x over previous
import functools

import jax
import jax.numpy as jnp
from jax import lax
from jax.experimental import pallas as pl
from jax.experimental.pallas import tpu as pltpu
from jax.experimental.pallas import tpu_sc as plsc

B, L, D = 4096, 200, 64
POS_V = 2 * 200 + 1
ACT_V = 8 + 1
NC, NS = 2, 16
NW = NC * NS
BPW = B // NW
CB = 16
NCB = BPW // CB
G0, G1 = 104, 96
NG = D // 16
PPT = 26
CPT = PPT * ACT_V
CV_PAD = NS * CPT
HPT = CPT // 2


def _sc_body(item_ids, pos_ids, act_ids, user_ids,
             item_tab, user_tab, pos_tab, act_tab,
             out_hbm,
             combo_sh, pos_loc, act_loc, uid_loc, user_loc,
             iid_loc, pid_loc, aid_loc, cidx, rows, combo,
             isem0, isem1, csem0, csem1, osem0, osem1, ssem):
    cid = lax.axis_index("c")
    sid = lax.axis_index("s")
    wid = sid * NC + cid
    base_b = wid * BPW

    pltpu.sync_copy(pos_tab, pos_loc.at[pl.ds(0, POS_V)])
    pltpu.sync_copy(act_tab, act_loc)
    pltpu.sync_copy(user_ids.at[pl.ds(base_b, BPW)], uid_loc)
    pltpu.async_copy(user_tab.at[uid_loc], user_loc, ssem).wait()

    avs = [[act_loc[a, pl.ds(g * 16, 16)] for g in range(NG)]
           for a in range(ACT_V)]
    p_base = sid * PPT
    for half in range(2):
        def build_p(pp, carry):
            p = p_base + half * (PPT // 2) + pp
            for g in range(NG):
                sl = pl.ds(g * 16, 16)
                pv = pos_loc[p, sl]
                for a in range(ACT_V):
                    rows[0, pp * ACT_V + a, sl] = pv + avs[a][g]
            return carry

        lax.fori_loop(0, PPT // 2, build_p, 0)
        pltpu.sync_copy(rows.at[0, pl.ds(0, HPT)],
                        combo_sh.at[pl.ds(sid * CPT + half * HPT, HPT)])
    plsc.subcore_barrier()

    isems = (isem0, isem1)
    csems = (csem0, csem1)
    osems = (osem0, osem1)

    def stage_ids(c):
        cb0 = base_b + c * CB
        cpar = lax.rem(c, 2)
        pltpu.sync_copy(item_ids.at[pl.ds(cb0, CB)], iid_loc.at[cpar])
        pltpu.sync_copy(pos_ids.at[pl.ds(cb0, CB)], pid_loc.at[cpar])
        pltpu.sync_copy(act_ids.at[pl.ds(cb0, CB)], aid_loc.at[cpar])

    def prep_cidx(b, par):
        cpar = lax.rem(b // CB, 2)
        bb = lax.rem(b, CB)
        for g16 in range(L // 16):
            i0 = g16 * 16
            sl = pl.ds(i0, 16)
            cidx[par, sl] = pid_loc[cpar, bb, sl] * ACT_V + aid_loc[cpar, bb, sl]
        sl = pl.ds(L - 16, 16)
        cidx[par, sl] = pid_loc[cpar, bb, sl] * ACT_V + aid_loc[cpar, bb, sl]

    def gathers(b, par):
        cpar = lax.rem(b // CB, 2)
        bb = lax.rem(b, CB)
        c0 = pltpu.async_copy(item_tab.at[iid_loc.at[cpar, bb, pl.ds(0, G0)]],
                              rows.at[par, pl.ds(0, G0)], isems[par])
        c1 = pltpu.async_copy(item_tab.at[iid_loc.at[cpar, bb, pl.ds(G0, G1)]],
                              rows.at[par, pl.ds(G0, G1)], isems[par])
        c2 = pltpu.async_copy(combo_sh.at[cidx.at[par, pl.ds(0, G0)]],
                              combo.at[par, pl.ds(0, G0)], csems[par])
        c3 = pltpu.async_copy(combo_sh.at[cidx.at[par, pl.ds(G0, G1)]],
                              combo.at[par, pl.ds(G0, G1)], csems[par])
        return c0, c1, c2, c3

    def wait_gathers(b, par):
        for h in gathers_desc(b, par):
            h.wait()

    def gathers_desc(b, par):
        cpar = lax.rem(b // CB, 2)
        bb = lax.rem(b, CB)
        return (
            pltpu.make_async_copy(item_tab.at[iid_loc.at[cpar, bb, pl.ds(0, G0)]],
                                  rows.at[par, pl.ds(0, G0)], isems[par]),
            pltpu.make_async_copy(item_tab.at[iid_loc.at[cpar, bb, pl.ds(G0, G1)]],
                                  rows.at[par, pl.ds(G0, G1)], isems[par]),
            pltpu.make_async_copy(combo_sh.at[cidx.at[par, pl.ds(0, G0)]],
                                  combo.at[par, pl.ds(0, G0)], csems[par]),
            pltpu.make_async_copy(combo_sh.at[cidx.at[par, pl.ds(G0, G1)]],
                                  combo.at[par, pl.ds(G0, G1)], csems[par]),
        )

    def issue_out(b, par):
        pltpu.async_copy(rows.at[par], out_hbm.at[base_b + b], osems[par])

    def wait_out(b, par):
        pltpu.make_async_copy(rows.at[par], out_hbm.at[base_b + b],
                              osems[par]).wait()

    def compute(b, par):
        uvs = [user_loc[b, pl.ds(g * 16, 16)] for g in range(NG)]

        def addrow(i, carry):
            for j in range(4):
                r = i * 4 + j
                for g in range(NG):
                    sl = pl.ds(g * 16, 16)
                    rows[par, r, sl] = (rows[par, r, sl] + combo[par, r, sl]
                                        + uvs[g])
            return carry

        lax.fori_loop(0, L // 4, addrow, 0)

    stage_ids(0)
    prep_cidx(0, 0)
    gathers(0, 0)

    def loop_body(k, carry):
        for par in range(2):
            b = 2 * k + par
            wait_gathers(b, par)
            nxt = b + 1

            @pl.when(nxt < BPW)
            def _prep():
                @pl.when(lax.rem(nxt, CB) == 0)
                def _stage():
                    stage_ids(nxt // CB)
                prep_cidx(nxt, 1 - par)

                @pl.when(b >= 1)
                def _wout():
                    wait_out(b - 1, 1 - par)
                gathers(nxt, 1 - par)

            compute(b, par)
            issue_out(b, par)
        return carry

    lax.fori_loop(0, BPW // 2, loop_body, 0)
    wait_out(BPW - 2, 0)
    wait_out(BPW - 1, 1)


@jax.jit
def _run(item_ids, pos_ids, act_ids, user_ids,
         item_tab, user_tab, pos_tab, act_tab):
    mesh = plsc.VectorSubcoreMesh(core_axis_name="c", subcore_axis_name="s")
    f = functools.partial(
        pl.kernel,
        out_type=jax.ShapeDtypeStruct((B, L, D), jnp.float32),
        mesh=mesh,
        compiler_params=pltpu.CompilerParams(use_tc_tiling_on_sc=False),
        scratch_types=[
            pltpu.VMEM_SHARED((CV_PAD, D), jnp.float32),
            pltpu.VMEM((NS * PPT, D), jnp.float32),
            pltpu.VMEM((ACT_V, D), jnp.float32),
            pltpu.VMEM((BPW,), jnp.int32),
            pltpu.VMEM((BPW, D), jnp.float32),
            pltpu.VMEM((2, CB, L), jnp.int32),
            pltpu.VMEM((2, CB, L), jnp.int32),
            pltpu.VMEM((2, CB, L), jnp.int32),
            pltpu.VMEM((2, L), jnp.int32),
            pltpu.VMEM((2, L, D), jnp.float32),
            pltpu.VMEM((2, L, D), jnp.float32),
            pltpu.SemaphoreType.DMA,
            pltpu.SemaphoreType.DMA,
            pltpu.SemaphoreType.DMA,
            pltpu.SemaphoreType.DMA,
            pltpu.SemaphoreType.DMA,
            pltpu.SemaphoreType.DMA,
            pltpu.SemaphoreType.DMA,
        ],
    )(_sc_body)
    return f(item_ids, pos_ids, act_ids, user_ids,
             item_tab, user_tab, pos_tab, act_tab)


def kernel(item_ids, pos_ids, action_ids, user_ids,
           item_table, user_table, pos_table, action_table):
    return _run(item_ids.astype(jnp.int32), pos_ids.astype(jnp.int32),
                action_ids.astype(jnp.int32), user_ids.astype(jnp.int32),
                item_table, user_table, pos_table, action_table)
